# baseline (speedup 1.0000x reference)
import functools

import jax
import jax.numpy as jnp
from jax import lax
from jax.experimental import pallas as pl
from jax.experimental.pallas import tpu as pltpu
from jax.experimental.pallas import tpu_sc as plsc

N = 10000
NP = 10240
E = 320000
EP = 331776
G = 16
NC = 2
NS = 16
L = 16
R = 2048
NB = NP // R
EPT = EP // NS
EPW = EP // (NC * NS)
CK = 128
STRIPE = NP // NS

_mesh = plsc.VectorSubcoreMesh(core_axis_name="c", subcore_axis_name="s")
_sc_params = pltpu.CompilerParams(needs_layout_passes=False,
                                  use_tc_tiling_on_sc=False)


def _iota16():
    return lax.iota(jnp.int32, L)


_GDN = lax.GatherDimensionNumbers(
    offset_dims=(), collapsed_slice_dims=(0,), start_index_map=(0,))


def _bcast_lane(v16, j2):
    idx = jnp.full((L, 1), j2, jnp.int32)
    return lax.gather(v16, idx, _GDN, (1,),
                      mode=lax.GatherScatterMode.PROMISE_IN_BOUNDS)


def _mm_body(x_ref, w_ref, b_ref, o_ref):
    o_ref[...] = (jnp.dot(x_ref[...], w_ref[...],
                          preferred_element_type=jnp.float32)
                  + b_ref[0]).astype(jnp.bfloat16)


def _matmul_sliced(x_pad, W, b_r, n_slices):
    return pl.pallas_call(
        _mm_body,
        grid=(n_slices, NB),
        in_specs=[
            pl.BlockSpec((R, 128), lambda i, j: (j, 0)),
            pl.BlockSpec((128, 128), lambda i, j: (0, i)),
            pl.BlockSpec((1, 1, 128), lambda i, j: (i, 0, 0)),
        ],
        out_specs=pl.BlockSpec((R, 128), lambda i, j: (i * NB + j, 0)),
        out_shape=jax.ShapeDtypeStruct((n_slices * NP, 128), jnp.bfloat16),
    )(x_pad, W, b_r)


def _newton_rsqrt(d):
    y = lax.bitcast_convert_type(
        jnp.full((L,), 0x5F3759DF, jnp.int32)
        - lax.shift_right_logical(lax.bitcast_convert_type(d, jnp.int32),
                                  jnp.full((L,), 1, jnp.int32)),
        jnp.float32)
    for _ in range(3):
        y = y * (1.5 - 0.5 * d * y * y)
    return y


def _prep_body(src_hbm, dst_hbm, w_hbm, norm_hbm,
               table, dtab, dstb, wb, dbuf, dv, sb, db, wbig, nb):
    c = lax.axis_index("c")
    s = lax.axis_index("s")

    def zz(jv, _):
        dbuf[pl.ds(jv * L, L)] = jnp.zeros((L,), jnp.float32)
        return 0
    lax.fori_loop(0, STRIPE // L, zz, 0)
    pltpu.sync_copy(dbuf, table.at[pl.ds(s * STRIPE, STRIPE)])
    plsc.subcore_barrier()

    def chunk(ch, _):
        base = s * EPT + ch * CK
        pltpu.sync_copy(dst_hbm.at[pl.ds(base, CK)], dstb)
        pltpu.sync_copy(w_hbm.at[pl.ds(base, CK)], wb)
        pltpu.sync_copy(wb, table.at[dstb], add=True)
        return 0
    lax.fori_loop(0, EPT // CK, chunk, 0)
    plsc.subcore_barrier()

    pltpu.sync_copy(table.at[pl.ds(s * STRIPE, STRIPE)], dbuf)

    def dstep(jv, _):
        dbuf[pl.ds(jv * L, L)] = _newton_rsqrt(dbuf[pl.ds(jv * L, L)])
        return 0
    lax.fori_loop(0, STRIPE // L, dstep, 0)
    pltpu.sync_copy(dbuf, dtab.at[pl.ds(s * STRIPE, STRIPE)])
    plsc.subcore_barrier()
    pltpu.sync_copy(dtab, dv)

    base = (c * NS + s) * EPW
    pltpu.sync_copy(src_hbm.at[pl.ds(base, EPW)], sb)
    pltpu.sync_copy(dst_hbm.at[pl.ds(base, EPW)], db)
    pltpu.sync_copy(w_hbm.at[pl.ds(base, EPW)], wbig)

    def step(j, _):
        s16 = sb[pl.ds(j * L, L)]
        d16 = db[pl.ds(j * L, L)]
        w16 = wbig[pl.ds(j * L, L)]
        n16 = plsc.load_gather(dv, [s16]) * w16 * plsc.load_gather(dv, [d16])
        nb[pl.ds(j * L, L)] = n16
        return 0
    lax.fori_loop(0, EPW // L, step, 0)
    pltpu.sync_copy(nb, norm_hbm.at[pl.ds(base, EPW)])


_prep_kernel = functools.partial(
    pl.kernel,
    out_type=jax.ShapeDtypeStruct((EP,), jnp.float32),
    mesh=_mesh,
    scratch_types=[
        pltpu.VMEM_SHARED((NP,), jnp.float32),
        pltpu.VMEM_SHARED((NP,), jnp.float32),
        pltpu.VMEM((CK,), jnp.int32),
        pltpu.VMEM((CK,), jnp.float32),
        pltpu.VMEM((STRIPE,), jnp.float32),
        pltpu.VMEM((NP,), jnp.float32),
        pltpu.VMEM((EPW,), jnp.int32),
        pltpu.VMEM((EPW,), jnp.int32),
        pltpu.VMEM((EPW,), jnp.float32),
        pltpu.VMEM((EPW,), jnp.float32),
    ],
    compiler_params=_sc_params,
)(_prep_body)


NCH = EPT // CK


def _fetch_chunk(pk_hbm, h_hbm, s, ch, off, pk, rbf, gs):
    pltpu.sync_copy(pk_hbm.at[s, ch], pk)
    for jv in range(CK // L):
        pk[0, pl.ds(jv * L, L)] = pk[0, pl.ds(jv * L, L)] + off
    pltpu.async_copy(h_hbm.at[pk.at[0]], rbf, gs)


def _spmm_body(n_passes, h_hbm, pk_hbm, out_hbm,
               acc, rbf0, rbf1, srows, pk0, pk1, db, gs0, gs1, ss):
    c = lax.axis_index("c")
    s = lax.axis_index("s")
    m16 = jnp.full((L,), jnp.int32(-65536))
    s16 = jnp.full((L,), 16, jnp.int32)

    def zr(j, _):
        for k in range(128 // L):
            srows[j, pl.ds(k * L, L)] = jnp.zeros((L,), jnp.float32)
        return 0

    for p in range(n_passes):
        sl = c * n_passes + p
        off = sl * NP
        lax.fori_loop(0, CK, zr, 0)
        for bq in range(STRIPE // CK):
            pltpu.sync_copy(srows, acc.at[pl.ds(s * STRIPE + bq * CK, CK)])
        plsc.subcore_barrier()

        _fetch_chunk(pk_hbm, h_hbm, s, 0, off, pk0, rbf0, gs0)

        def do_chunk(ch, rbf_c, pk_c, gs_c, rbf_n, pk_n, gs_n):
            @pl.when(ch + 1 < NCH)
            def _():
                _fetch_chunk(pk_hbm, h_hbm, s, ch + 1, off, pk_n, rbf_n, gs_n)
            pltpu.make_async_copy(h_hbm.at[pk_c.at[0]], rbf_c, gs_c).wait()

            @pl.when(ch >= 1)
            def _():
                pltpu.make_async_copy(srows, acc.at[db], ss).wait()

            def scale(jv, _):
                n16 = lax.bitcast_convert_type(pk_c[2, pl.ds(jv * L, L)],
                                               jnp.float32)
                db[pl.ds(jv * L, L)] = pk_c[1, pl.ds(jv * L, L)]
                for j2 in range(L):
                    bc = _bcast_lane(n16, j2)
                    j = jv * L + j2
                    for k in range(128 // (2 * L)):
                        v = rbf_c[j, pl.ds(k * L, L)]
                        a = lax.bitcast_convert_type(
                            lax.shift_left(v, s16), jnp.float32)
                        b = lax.bitcast_convert_type(v & m16, jnp.float32)
                        srows[j, pl.ds(k * 2 * L, L)] = a * bc
                        srows[j, pl.ds(k * 2 * L + L, L)] = b * bc
                return 0
            lax.fori_loop(0, CK // L, scale, 0)
            pltpu.async_copy(srows, acc.at[db], ss, add=True)

        def pair(ch2, _):
            do_chunk(2 * ch2, rbf0, pk0, gs0, rbf1, pk1, gs1)
            do_chunk(2 * ch2 + 1, rbf1, pk1, gs1, rbf0, pk0, gs0)
            return 0
        lax.fori_loop(0, NCH // 2, pair, 0)
        pltpu.make_async_copy(srows, acc.at[db], ss).wait()
        plsc.subcore_barrier()

        for bq in range(STRIPE // CK):
            rb = s * STRIPE + bq * CK
            pltpu.sync_copy(acc.at[pl.ds(rb, CK)],
                            out_hbm.at[pl.ds(off + rb, CK)])
        if p + 1 < n_passes:
            plsc.subcore_barrier()


def _spmm(h_flat, packed, n_slices):
    n_passes = n_slices // NC
    body = functools.partial(_spmm_body, n_passes)
    return pl.kernel(
        body,
        out_type=jax.ShapeDtypeStruct((n_slices * NP, 128), jnp.float32),
        mesh=_mesh,
        scratch_types=[
            pltpu.VMEM_SHARED((NP, 128), jnp.float32),
            pltpu.VMEM((CK, 64), jnp.int32),
            pltpu.VMEM((CK, 64), jnp.int32),
            pltpu.VMEM((CK, 128), jnp.float32),
            pltpu.VMEM((3, CK), jnp.int32),
            pltpu.VMEM((3, CK), jnp.int32),
            pltpu.VMEM((CK,), jnp.int32),
            pltpu.SemaphoreType.DMA,
            pltpu.SemaphoreType.DMA,
            pltpu.SemaphoreType.DMA,
        ],
        compiler_params=_sc_params,
    )(h_flat, packed)


def _stats_body(n_slices, o_ref, g_ref, b_ref, ss_ref):
    j = pl.program_id(0)

    @pl.when(j == 0)
    def _():
        ss_ref[...] = jnp.zeros_like(ss_ref)

    blk = o_ref[...]
    ss_ref[0] += jnp.sum(blk, axis=1)
    ss_ref[1] += jnp.sum(blk * blk, axis=1)

    @pl.when(j == NB - 1)
    def _():
        mu = ss_ref[0] / float(N)
        var = ss_ref[1] / float(N) - mu * mu
        scale = g_ref[...] * lax.rsqrt(var + 1e-5)
        ss_ref[0] = scale
        ss_ref[1] = b_ref[...] - mu * scale


def _bn_stats(out_flat, g, b, n_slices):
    body = functools.partial(_stats_body, n_slices)
    return pl.pallas_call(
        body,
        grid=(NB,),
        in_specs=[
            pl.BlockSpec((n_slices, R, 128), lambda j: (0, j, 0)),
            pl.BlockSpec((n_slices, 128), lambda j: (0, 0)),
            pl.BlockSpec((n_slices, 128), lambda j: (0, 0)),
        ],
        out_specs=pl.BlockSpec((2, n_slices, 128), lambda j: (0, 0, 0)),
        out_shape=jax.ShapeDtypeStruct((2, n_slices, 128), jnp.float32),
    )(out_flat.reshape(n_slices, NP, 128), g.reshape(n_slices, 128),
      b.reshape(n_slices, 128))


def _bnmm_body(o_ref, ss_ref, w_ref, b_ref, h_ref):
    a0 = jnp.maximum(o_ref[0] * ss_ref[0, 0] + ss_ref[1, 0], 0.0)
    a1 = jnp.maximum(o_ref[1] * ss_ref[0, 1] + ss_ref[1, 1], 0.0)
    w = w_ref[...]
    h_ref[...] = (jnp.dot(a0, w[0:128], preferred_element_type=jnp.float32)
                  + jnp.dot(a1, w[128:256], preferred_element_type=jnp.float32)
                  + b_ref[0]).astype(jnp.bfloat16)


def _bn_relu_matmul(out_flat, ss, W, b_r, n_out_slices):
    return pl.pallas_call(
        _bnmm_body,
        grid=(n_out_slices, NB),
        in_specs=[
            pl.BlockSpec((2, R, 128), lambda i, j: (0, j, 0)),
            pl.BlockSpec((2, 2, 128), lambda i, j: (0, 0, 0)),
            pl.BlockSpec((256, 128), lambda i, j: (0, i)),
            pl.BlockSpec((1, 1, 128), lambda i, j: (i, 0, 0)),
        ],
        out_specs=pl.BlockSpec((R, 128), lambda i, j: (i * NB + j, 0)),
        out_shape=jax.ShapeDtypeStruct((n_out_slices * NP, 128), jnp.bfloat16),
    )(out_flat.reshape(2, NP, 128), ss, W, b_r)


def _final_body(o_ref, ss_ref, bt_ref, w2_ref, b2_ref, w1_ref, b1_ref,
                w0_ref, b0_ref, out_ref, pool_ref):
    j = pl.program_id(0)

    @pl.when(j == 0)
    def _():
        pool_ref[...] = jnp.zeros_like(pool_ref)

    acts = [jnp.maximum(o_ref[i] * ss_ref[0, i] + ss_ref[1, i], 0.0)
            for i in range(4)]
    acts.append(jnp.ones((R, 128), jnp.float32))
    act = jnp.concatenate(acts, axis=1)
    lanes = lax.broadcasted_iota(jnp.int32, (R, 128), 1).astype(jnp.float32)
    oh = (bt_ref[...] == lanes).astype(jnp.float32)
    pool_ref[...] += lax.dot_general(oh, act, (((0,), (0,)), ((), ())))

    @pl.when(j == NB - 1)
    def _():
        pooled = pool_ref[...]
        cnt = jnp.maximum(pooled[:, 512:513], 1.0)
        mean = pooled[:, 0:512] / cnt
        h = jnp.maximum(jnp.dot(mean, w2_ref[...],
                                preferred_element_type=jnp.float32)
                        + b2_ref[...], 0.0)
        h = jnp.maximum(jnp.dot(h, w1_ref[...],
                                preferred_element_type=jnp.float32)
                        + b1_ref[...], 0.0)
        h = jnp.maximum(jnp.dot(h, w0_ref[...],
                                preferred_element_type=jnp.float32)
                        + b0_ref[...], 0.0)
        out_ref[...] = h[0:G, :]


def _final(out_flat, ss, batch_bcast, Wl2, bl2, Wl1, bl1, Wl0, bl0):
    return pl.pallas_call(
        _final_body,
        grid=(NB,),
        in_specs=[
            pl.BlockSpec((4, R, 128), lambda j: (0, j, 0)),
            pl.BlockSpec((2, 4, 128), lambda j: (0, 0, 0)),
            pl.BlockSpec((R, 128), lambda j: (j, 0)),
            pl.BlockSpec((512, 256), lambda j: (0, 0)),
            pl.BlockSpec((1, 256), lambda j: (0, 0)),
            pl.BlockSpec((256, 128), lambda j: (0, 0)),
            pl.BlockSpec((1, 128), lambda j: (0, 0)),
            pl.BlockSpec((128, 64), lambda j: (0, 0)),
            pl.BlockSpec((1, 64), lambda j: (0, 0)),
        ],
        out_specs=pl.BlockSpec((G, 64), lambda j: (0, 0)),
        out_shape=jax.ShapeDtypeStruct((G, 64), jnp.float32),
        scratch_shapes=[pltpu.VMEM((128, 640), jnp.float32)],
    )(out_flat.reshape(4, NP, 128), ss, batch_bcast,
      Wl2, bl2.reshape(1, -1), Wl1, bl1.reshape(1, -1), Wl0, bl0.reshape(1, -1))


def _interleave_cols(D):
    j = jnp.arange(D)
    return (j // 32) * 32 + (j % 2) * 16 + (j % 32) // 2


def kernel(x, edge_index, edge_weight, batch,
           Wc0, bc0, g0, be0, Wc1, bc1, g1, be1,
           Wl2, bl2, Wl1, bl1, Wl0, bl0):
    loop = jnp.arange(N, dtype=jnp.int32)
    pad = EP - E - N
    srcb = jnp.concatenate([edge_index[0], loop,
                            jnp.zeros((pad,), jnp.int32)])
    dstb = jnp.concatenate([edge_index[1], loop,
                            jnp.zeros((pad,), jnp.int32)])
    wb = jnp.concatenate([edge_weight, jnp.ones((N,), jnp.float32),
                          jnp.zeros((pad,), jnp.float32)])
    x_pad = jnp.pad(x, ((0, NP - N), (0, 0)))
    batch_bcast = jnp.broadcast_to(
        jnp.pad(batch, (0, NP - N), constant_values=G).astype(jnp.float32)[:, None],
        (NP, 128))

    p256 = _interleave_cols(256)
    p512 = _interleave_cols(512)
    h0 = _matmul_sliced(x_pad, Wc0[:, p256], bc0[p256].reshape(2, 1, 128), 2)
    norm = _prep_kernel(srcb, dstb, wb)
    packed = jnp.stack(
        [srcb.reshape(NS, NCH, CK), dstb.reshape(NS, NCH, CK),
         lax.bitcast_convert_type(norm, jnp.int32).reshape(NS, NCH, CK)],
        axis=2)
    h0p = lax.bitcast_convert_type(h0.reshape(2 * NP, 64, 2), jnp.int32)
    out0 = _spmm(h0p, packed, 2)
    ss0 = _bn_stats(out0, g0, be0, 2)

    h1 = _bn_relu_matmul(out0, ss0, Wc1[:, p512], bc1[p512].reshape(4, 1, 128), 4)
    h1p = lax.bitcast_convert_type(h1.reshape(4 * NP, 64, 2), jnp.int32)
    out1 = _spmm(h1p, packed, 4)
    ss1 = _bn_stats(out1, g1, be1, 4)

    return _final(out1, ss1, batch_bcast, Wl2, bl2, Wl1, bl1, Wl0, bl0)


# speedup vs baseline: 1.5980x; 1.5980x over previous
import functools

import jax
import jax.numpy as jnp
from jax import lax
from jax.experimental import pallas as pl
from jax.experimental.pallas import tpu as pltpu
from jax.experimental.pallas import tpu_sc as plsc

N = 10000
NP = 10240
E = 320000
EP = 331776
G = 16
NC = 2
NS = 16
L = 16
R = 2048
NB = NP // R
EPT = EP // NS
EPW = EP // (NC * NS)
CK = 128
STRIPE = NP // NS

_mesh = plsc.VectorSubcoreMesh(core_axis_name="c", subcore_axis_name="s")
_sc_params = pltpu.CompilerParams(needs_layout_passes=False)


def _iota16():
    return lax.iota(jnp.int32, L)


_GDN = lax.GatherDimensionNumbers(
    offset_dims=(), collapsed_slice_dims=(0,), start_index_map=(0,))


def _bcast_lane(v16, j2):
    idx = jnp.full((L, 1), j2, jnp.int32)
    return lax.gather(v16, idx, _GDN, (1,),
                      mode=lax.GatherScatterMode.PROMISE_IN_BOUNDS)


def _mm_body(x_ref, w_ref, b_ref, o_ref):
    o_ref[...] = jnp.dot(x_ref[...], w_ref[...],
                         preferred_element_type=jnp.float32) + b_ref[0]


def _matmul_sliced(x_pad, W, b_r, n_slices):
    return pl.pallas_call(
        _mm_body,
        grid=(n_slices, NB),
        in_specs=[
            pl.BlockSpec((R, 128), lambda i, j: (j, 0)),
            pl.BlockSpec((128, 128), lambda i, j: (0, i)),
            pl.BlockSpec((1, 1, 128), lambda i, j: (i, 0, 0)),
        ],
        out_specs=pl.BlockSpec((R, 128), lambda i, j: (i * NB + j, 0)),
        out_shape=jax.ShapeDtypeStruct((n_slices * NP, 128), jnp.float32),
    )(x_pad, W, b_r)


def _newton_rsqrt(d):
    y = lax.bitcast_convert_type(
        jnp.full((L,), 0x5F3759DF, jnp.int32)
        - lax.shift_right_logical(lax.bitcast_convert_type(d, jnp.int32),
                                  jnp.full((L,), 1, jnp.int32)),
        jnp.float32)
    for _ in range(3):
        y = y * (1.5 - 0.5 * d * y * y)
    return y


def _prep_body(src_hbm, dst_hbm, w_hbm, norm_hbm,
               table, dtab, dstb, wb, dbuf, dv, sb, db, wbig, nb):
    c = lax.axis_index("c")
    s = lax.axis_index("s")

    def zz(jv, _):
        dbuf[pl.ds(jv * L, L)] = jnp.zeros((L,), jnp.float32)
        return 0
    lax.fori_loop(0, STRIPE // L, zz, 0)
    pltpu.sync_copy(dbuf, table.at[pl.ds(s * STRIPE, STRIPE)])
    plsc.subcore_barrier()

    def chunk(ch, _):
        base = s * EPT + ch * CK
        pltpu.sync_copy(dst_hbm.at[pl.ds(base, CK)], dstb)
        pltpu.sync_copy(w_hbm.at[pl.ds(base, CK)], wb)
        pltpu.sync_copy(wb, table.at[dstb], add=True)
        return 0
    lax.fori_loop(0, EPT // CK, chunk, 0)
    plsc.subcore_barrier()

    pltpu.sync_copy(table.at[pl.ds(s * STRIPE, STRIPE)], dbuf)

    def dstep(jv, _):
        dbuf[pl.ds(jv * L, L)] = _newton_rsqrt(dbuf[pl.ds(jv * L, L)])
        return 0
    lax.fori_loop(0, STRIPE // L, dstep, 0)
    pltpu.sync_copy(dbuf, dtab.at[pl.ds(s * STRIPE, STRIPE)])
    plsc.subcore_barrier()
    pltpu.sync_copy(dtab, dv)

    base = (c * NS + s) * EPW
    pltpu.sync_copy(src_hbm.at[pl.ds(base, EPW)], sb)
    pltpu.sync_copy(dst_hbm.at[pl.ds(base, EPW)], db)
    pltpu.sync_copy(w_hbm.at[pl.ds(base, EPW)], wbig)

    def step(j, _):
        s16 = sb[pl.ds(j * L, L)]
        d16 = db[pl.ds(j * L, L)]
        w16 = wbig[pl.ds(j * L, L)]
        n16 = plsc.load_gather(dv, [s16]) * w16 * plsc.load_gather(dv, [d16])
        nb[pl.ds(j * L, L)] = n16
        return 0
    lax.fori_loop(0, EPW // L, step, 0)
    pltpu.sync_copy(nb, norm_hbm.at[pl.ds(base, EPW)])


_prep_kernel = functools.partial(
    pl.kernel,
    out_type=jax.ShapeDtypeStruct((EP,), jnp.float32),
    mesh=_mesh,
    scratch_types=[
        pltpu.VMEM_SHARED((NP,), jnp.float32),
        pltpu.VMEM_SHARED((NP,), jnp.float32),
        pltpu.VMEM((CK,), jnp.int32),
        pltpu.VMEM((CK,), jnp.float32),
        pltpu.VMEM((STRIPE,), jnp.float32),
        pltpu.VMEM((NP,), jnp.float32),
        pltpu.VMEM((EPW,), jnp.int32),
        pltpu.VMEM((EPW,), jnp.int32),
        pltpu.VMEM((EPW,), jnp.float32),
        pltpu.VMEM((EPW,), jnp.float32),
    ],
    compiler_params=_sc_params,
)(_prep_body)


NCH = EPT // CK


def _fetch_chunk(pk_hbm, h_hbm, acc, s, ch, off, pk, rows, gs, ss):
    @pl.when(ch >= 2)
    def _():
        pltpu.make_async_copy(rows, acc.at[pk.at[1]], ss).wait()
    pltpu.sync_copy(pk_hbm.at[s, ch], pk)
    for jv in range(CK // L):
        pk[0, pl.ds(jv * L, L)] = pk[0, pl.ds(jv * L, L)] + off
    pltpu.async_copy(h_hbm.at[pk.at[0]], rows, gs)


def _spmm_body(n_passes, h_hbm, pk_hbm, out_hbm,
               acc, rows0, rows1, pk0, pk1, gs0, gs1, ss0, ss1):
    c = lax.axis_index("c")
    s = lax.axis_index("s")

    def zr(j, _):
        for k in range(128 // L):
            rows0[j, pl.ds(k * L, L)] = jnp.zeros((L,), jnp.float32)
        return 0

    for p in range(n_passes):
        sl = c * n_passes + p
        off = sl * NP
        lax.fori_loop(0, CK, zr, 0)
        for bq in range(STRIPE // CK):
            pltpu.sync_copy(rows0, acc.at[pl.ds(s * STRIPE + bq * CK, CK)])
        plsc.subcore_barrier()

        _fetch_chunk(pk_hbm, h_hbm, acc, s, 0, off, pk0, rows0, gs0, ss0)

        def do_chunk(ch, rows_c, pk_c, gs_c, ss_c, rows_n, pk_n, gs_n, ss_n):
            @pl.when(ch + 1 < NCH)
            def _():
                _fetch_chunk(pk_hbm, h_hbm, acc, s, ch + 1, off,
                             pk_n, rows_n, gs_n, ss_n)
            pltpu.make_async_copy(h_hbm.at[pk_c.at[0]], rows_c, gs_c).wait()

            def scale(jv, _):
                n16 = lax.bitcast_convert_type(pk_c[2, pl.ds(jv * L, L)],
                                               jnp.float32)
                for j2 in range(L):
                    bc = _bcast_lane(n16, j2)
                    for k in range(128 // L):
                        rows_c[jv * L + j2, pl.ds(k * L, L)] = (
                            rows_c[jv * L + j2, pl.ds(k * L, L)] * bc)
                return 0
            lax.fori_loop(0, CK // L, scale, 0)
            pltpu.async_copy(rows_c, acc.at[pk_c.at[1]], ss_c, add=True)

        def pair(ch2, _):
            do_chunk(2 * ch2, rows0, pk0, gs0, ss0, rows1, pk1, gs1, ss1)
            do_chunk(2 * ch2 + 1, rows1, pk1, gs1, ss1, rows0, pk0, gs0, ss0)
            return 0
        lax.fori_loop(0, NCH // 2, pair, 0)
        pltpu.make_async_copy(rows0, acc.at[pk0.at[1]], ss0).wait()
        pltpu.make_async_copy(rows1, acc.at[pk1.at[1]], ss1).wait()
        plsc.subcore_barrier()

        for bq in range(STRIPE // CK):
            rb = s * STRIPE + bq * CK
            pltpu.sync_copy(acc.at[pl.ds(rb, CK)],
                            out_hbm.at[pl.ds(off + rb, CK)])
        if p + 1 < n_passes:
            plsc.subcore_barrier()


def _spmm(h_flat, packed, n_slices):
    n_passes = n_slices // NC
    body = functools.partial(_spmm_body, n_passes)
    return pl.kernel(
        body,
        out_type=jax.ShapeDtypeStruct((n_slices * NP, 128), jnp.float32),
        mesh=_mesh,
        scratch_types=[
            pltpu.VMEM_SHARED((NP, 128), jnp.float32),
            pltpu.VMEM((CK, 128), jnp.float32),
            pltpu.VMEM((CK, 128), jnp.float32),
            pltpu.VMEM((3, CK), jnp.int32),
            pltpu.VMEM((3, CK), jnp.int32),
            pltpu.SemaphoreType.DMA,
            pltpu.SemaphoreType.DMA,
            pltpu.SemaphoreType.DMA,
            pltpu.SemaphoreType.DMA,
        ],
        compiler_params=_sc_params,
    )(h_flat, packed)


def _stats_body(n_slices, o_ref, g_ref, b_ref, ss_ref):
    j = pl.program_id(0)

    @pl.when(j == 0)
    def _():
        ss_ref[...] = jnp.zeros_like(ss_ref)

    blk = o_ref[...]
    ss_ref[0] += jnp.sum(blk, axis=1)
    ss_ref[1] += jnp.sum(blk * blk, axis=1)

    @pl.when(j == NB - 1)
    def _():
        mu = ss_ref[0] / float(N)
        var = ss_ref[1] / float(N) - mu * mu
        scale = g_ref[...] * lax.rsqrt(var + 1e-5)
        ss_ref[0] = scale
        ss_ref[1] = b_ref[...] - mu * scale


def _bn_stats(out_flat, g, b, n_slices):
    body = functools.partial(_stats_body, n_slices)
    return pl.pallas_call(
        body,
        grid=(NB,),
        in_specs=[
            pl.BlockSpec((n_slices, R, 128), lambda j: (0, j, 0)),
            pl.BlockSpec((n_slices, 128), lambda j: (0, 0)),
            pl.BlockSpec((n_slices, 128), lambda j: (0, 0)),
        ],
        out_specs=pl.BlockSpec((2, n_slices, 128), lambda j: (0, 0, 0)),
        out_shape=jax.ShapeDtypeStruct((2, n_slices, 128), jnp.float32),
    )(out_flat.reshape(n_slices, NP, 128), g.reshape(n_slices, 128),
      b.reshape(n_slices, 128))


def _bnmm_body(o_ref, ss_ref, w_ref, b_ref, h_ref):
    a0 = jnp.maximum(o_ref[0] * ss_ref[0, 0] + ss_ref[1, 0], 0.0)
    a1 = jnp.maximum(o_ref[1] * ss_ref[0, 1] + ss_ref[1, 1], 0.0)
    w = w_ref[...]
    h_ref[...] = (jnp.dot(a0, w[0:128], preferred_element_type=jnp.float32)
                  + jnp.dot(a1, w[128:256], preferred_element_type=jnp.float32)
                  + b_ref[0])


def _bn_relu_matmul(out_flat, ss, W, b_r, n_out_slices):
    return pl.pallas_call(
        _bnmm_body,
        grid=(n_out_slices, NB),
        in_specs=[
            pl.BlockSpec((2, R, 128), lambda i, j: (0, j, 0)),
            pl.BlockSpec((2, 2, 128), lambda i, j: (0, 0, 0)),
            pl.BlockSpec((256, 128), lambda i, j: (0, i)),
            pl.BlockSpec((1, 1, 128), lambda i, j: (i, 0, 0)),
        ],
        out_specs=pl.BlockSpec((R, 128), lambda i, j: (i * NB + j, 0)),
        out_shape=jax.ShapeDtypeStruct((n_out_slices * NP, 128), jnp.float32),
    )(out_flat.reshape(2, NP, 128), ss, W, b_r)


def _final_body(o_ref, ss_ref, bt_ref, w2_ref, b2_ref, w1_ref, b1_ref,
                w0_ref, b0_ref, out_ref, pool_ref):
    j = pl.program_id(0)

    @pl.when(j == 0)
    def _():
        pool_ref[...] = jnp.zeros_like(pool_ref)

    acts = [jnp.maximum(o_ref[i] * ss_ref[0, i] + ss_ref[1, i], 0.0)
            for i in range(4)]
    acts.append(jnp.ones((R, 128), jnp.float32))
    act = jnp.concatenate(acts, axis=1)
    lanes = lax.broadcasted_iota(jnp.int32, (R, 128), 1).astype(jnp.float32)
    oh = (bt_ref[...] == lanes).astype(jnp.float32)
    pool_ref[...] += lax.dot_general(oh, act, (((0,), (0,)), ((), ())))

    @pl.when(j == NB - 1)
    def _():
        pooled = pool_ref[...]
        cnt = jnp.maximum(pooled[:, 512:513], 1.0)
        mean = pooled[:, 0:512] / cnt
        h = jnp.maximum(jnp.dot(mean, w2_ref[...],
                                preferred_element_type=jnp.float32)
                        + b2_ref[...], 0.0)
        h = jnp.maximum(jnp.dot(h, w1_ref[...],
                                preferred_element_type=jnp.float32)
                        + b1_ref[...], 0.0)
        h = jnp.maximum(jnp.dot(h, w0_ref[...],
                                preferred_element_type=jnp.float32)
                        + b0_ref[...], 0.0)
        out_ref[...] = h[0:G, :]


def _final(out_flat, ss, batch_bcast, Wl2, bl2, Wl1, bl1, Wl0, bl0):
    return pl.pallas_call(
        _final_body,
        grid=(NB,),
        in_specs=[
            pl.BlockSpec((4, R, 128), lambda j: (0, j, 0)),
            pl.BlockSpec((2, 4, 128), lambda j: (0, 0, 0)),
            pl.BlockSpec((R, 128), lambda j: (j, 0)),
            pl.BlockSpec((512, 256), lambda j: (0, 0)),
            pl.BlockSpec((1, 256), lambda j: (0, 0)),
            pl.BlockSpec((256, 128), lambda j: (0, 0)),
            pl.BlockSpec((1, 128), lambda j: (0, 0)),
            pl.BlockSpec((128, 64), lambda j: (0, 0)),
            pl.BlockSpec((1, 64), lambda j: (0, 0)),
        ],
        out_specs=pl.BlockSpec((G, 64), lambda j: (0, 0)),
        out_shape=jax.ShapeDtypeStruct((G, 64), jnp.float32),
        scratch_shapes=[pltpu.VMEM((128, 640), jnp.float32)],
    )(out_flat.reshape(4, NP, 128), ss, batch_bcast,
      Wl2, bl2.reshape(1, -1), Wl1, bl1.reshape(1, -1), Wl0, bl0.reshape(1, -1))


def kernel(x, edge_index, edge_weight, batch,
           Wc0, bc0, g0, be0, Wc1, bc1, g1, be1,
           Wl2, bl2, Wl1, bl1, Wl0, bl0):
    loop = jnp.arange(N, dtype=jnp.int32)
    pad = EP - E - N
    srcb = jnp.concatenate([edge_index[0], loop,
                            jnp.zeros((pad,), jnp.int32)])
    dstb = jnp.concatenate([edge_index[1], loop,
                            jnp.zeros((pad,), jnp.int32)])
    wb = jnp.concatenate([edge_weight, jnp.ones((N,), jnp.float32),
                          jnp.zeros((pad,), jnp.float32)])
    x_pad = jnp.pad(x, ((0, NP - N), (0, 0)))
    batch_bcast = jnp.broadcast_to(
        jnp.pad(batch, (0, NP - N), constant_values=G).astype(jnp.float32)[:, None],
        (NP, 128))

    h0 = _matmul_sliced(x_pad, Wc0, bc0.reshape(2, 1, 128), 2)
    norm = _prep_kernel(srcb, dstb, wb)
    packed = jnp.stack(
        [srcb.reshape(NS, NCH, CK), dstb.reshape(NS, NCH, CK),
         lax.bitcast_convert_type(norm, jnp.int32).reshape(NS, NCH, CK)],
        axis=2)
    out0 = _spmm(h0, packed, 2)
    ss0 = _bn_stats(out0, g0, be0, 2)

    h1 = _bn_relu_matmul(out0, ss0, Wc1, bc1.reshape(4, 1, 128), 4)
    out1 = _spmm(h1, packed, 4)
    ss1 = _bn_stats(out1, g1, be1, 4)

    return _final(out1, ss1, batch_bcast, Wl2, bl2, Wl1, bl1, Wl0, bl0)
